# baseline (speedup 1.0000x reference)
import functools

import jax
import jax.numpy as jnp
import numpy as np
from jax import lax
from jax.experimental import pallas as pl
from jax.experimental.pallas import tpu as pltpu
from jax.experimental.pallas import tpu_sc as plsc

NC = 2
NS = 16
NW = NC * NS
CH = 128
NBUF = 8


def _sc_mesh():
    return plsc.VectorSubcoreMesh(core_axis_name="c", subcore_axis_name="s")


def _deg_kernel(nc0, nc1, nacc, dst_hbm, ones_hbm, zeros_hbm, out_hbm,
                dst_v, ones_v, accum):
    cid = lax.axis_index("c")
    sid = lax.axis_index("s")
    wid = sid * NC + cid
    nchunk = lax.select(cid == 0, jnp.int32(nc0), jnp.int32(nc1))
    rows = nacc // NS
    pltpu.sync_copy(dst_hbm.at[wid], dst_v)
    pltpu.sync_copy(ones_hbm, ones_v)
    pltpu.sync_copy(zeros_hbm, accum.at[pl.ds(sid * rows, rows)])
    plsc.subcore_barrier()

    def body(j, carry):
        pltpu.sync_copy(ones_v, accum.at[dst_v.at[j]], add=True)
        return carry

    lax.fori_loop(0, nchunk, body, 0)
    plsc.subcore_barrier()
    pltpu.sync_copy(accum.at[pl.ds(sid * rows, rows)],
                    out_hbm.at[cid, pl.ds(sid * rows, rows)])


def _agg_kernel(nc0, nc1, nacc, hp_hbm, src_hbm, dst_hbm, zeros_hbm, out_hbm,
                src_v, dst_v, *scratch):
    cid = lax.axis_index("c")
    sid = lax.axis_index("s")
    wid = sid * NC + cid
    nchunk = lax.select(cid == 0, jnp.int32(nc0), jnp.int32(nc1))
    rows = nacc // NS
    bufs = scratch[:NBUF]
    accum = scratch[NBUF]
    gsem = scratch[NBUF + 1:2 * NBUF + 1]
    ssem = scratch[2 * NBUF + 1:]
    pltpu.sync_copy(src_hbm.at[wid], src_v)
    pltpu.sync_copy(dst_hbm.at[wid], dst_v)
    pltpu.sync_copy(zeros_hbm, accum.at[pl.ds(sid * rows, rows)])
    plsc.subcore_barrier()

    for b in range(NBUF):
        pltpu.async_copy(hp_hbm.at[src_v.at[b]], bufs[b], gsem[b])

    def body(i, carry):
        base = i * NBUF
        for b in range(NBUF):
            j = base + b
            pltpu.make_async_copy(hp_hbm.at[src_v.at[j]], bufs[b],
                                  gsem[b]).wait()
            pltpu.async_copy(bufs[b], accum.at[dst_v.at[j]], ssem[b],
                             add=True)
        for b in range(NBUF):
            j = base + b
            jn = j + NBUF
            pltpu.make_async_copy(bufs[b], accum.at[dst_v.at[j]],
                                  ssem[b]).wait()

            @pl.when(jn < nchunk)
            def _():
                pltpu.async_copy(hp_hbm.at[src_v.at[jn]], bufs[b], gsem[b])

        return carry

    lax.fori_loop(0, nchunk // NBUF, body, 0)
    plsc.subcore_barrier()
    pltpu.sync_copy(accum.at[pl.ds(sid * rows, rows)],
                    out_hbm.at[cid, pl.ds(sid * rows, rows)])


def _stage_a_body(n, x_ref, w1_ref, d0_ref, d1_ref, o_ref):
    deg = d0_ref[:, 0:1] + d1_ref[:, 0:1] + 1.0
    dis = lax.rsqrt(deg[:n])
    hw = jnp.dot(x_ref[...], w1_ref[...], preferred_element_type=jnp.float32)
    o_ref[...] = hw * dis


def _stage_b_body(n, a0_ref, a1_ref, hp_ref, d0_ref, d1_ref, b1_ref, w2_ref,
                  o_ref):
    deg = d0_ref[:, 0:1] + d1_ref[:, 0:1] + 1.0
    dis = lax.rsqrt(deg[:n])
    agg = a0_ref[:n] + a1_ref[:n] + hp_ref[...]
    h1 = jax.nn.relu(agg * dis + b1_ref[...])
    o_ref[...] = jnp.dot(h1, w2_ref[...], preferred_element_type=jnp.float32) * dis


def _stage_c_body(n, g, ptr_ref, a0_ref, a1_ref, hp_ref, d0_ref, d1_ref,
                  b2_ref, wp_ref, bp_ref, o_ref):
    deg = d0_ref[:, 0:1] + d1_ref[:, 0:1] + 1.0
    dis = lax.rsqrt(deg[:n])
    agg = a0_ref[:n] + a1_ref[:n] + hp_ref[...]
    h2 = jax.nn.relu(agg * dis + b2_ref[...])
    idx = lax.broadcasted_iota(jnp.int32, (n, 1), 0)
    means = []
    for gi in range(g):
        lo = ptr_ref[gi]
        hi = ptr_ref[gi + 1]
        m = (idx >= lo) & (idx < hi)
        s = jnp.sum(jnp.where(m, h2, 0.0), axis=0, keepdims=True)
        cnt = jnp.maximum((hi - lo).astype(jnp.float32), 1.0)
        means.append(s / cnt)
    mean = jnp.concatenate(means, axis=0)
    o_ref[...] = jnp.dot(mean, wp_ref[...],
                         preferred_element_type=jnp.float32) + bp_ref[...]


def kernel(x, edge_index, ptr, W1, b1, W2, b2, Wp, bp):
    n, d = x.shape
    h = W1.shape[1]
    g = ptr.shape[0] - 1
    e = edge_index.shape[1]

    F0 = 0.72
    ept0 = max(int(e * F0 / NS) // (CH * NBUF) * (CH * NBUF), CH * NBUF)
    e1 = max(e - NS * ept0, 0)
    ept1 = max(-(-(-(-e1 // NS)) // (CH * NBUF)) * (CH * NBUF), CH * NBUF)
    nc0, nc1 = ept0 // CH, ept1 // CH
    ncmax = max(nc0, nc1)
    rows_per_tile = ((-(-n // NS)) + 7) // 8 * 8
    nacc = NS * rows_per_tile

    perm = np.full((NW, ncmax * CH), e, np.int64)
    pos = 0
    for w in range(NW):
        cap = ept0 if w % NC == 0 else ept1
        take = min(cap, e - pos)
        perm[w, :take] = np.arange(pos, pos + take)
        pos += take
    assert pos == e, (pos, e)
    perm = jnp.asarray(perm.reshape(NW, ncmax, CH), dtype=jnp.int32)

    src = jnp.take(jnp.concatenate([edge_index[0], jnp.zeros((1,), jnp.int32)]),
                   perm)
    dst = jnp.take(jnp.concatenate([edge_index[1], jnp.full((1,), n, jnp.int32)]),
                   perm)

    ones16 = jnp.ones((CH, 16), jnp.float32)
    zeros16 = jnp.zeros((rows_per_tile, 16), jnp.float32)
    zerosh = jnp.zeros((rows_per_tile, h), jnp.float32)

    deg_call = pl.kernel(
        functools.partial(_deg_kernel, nc0, nc1, nacc),
        out_type=jax.ShapeDtypeStruct((NC, nacc, 16), jnp.float32),
        mesh=_sc_mesh(),
        scratch_types=[
            pltpu.VMEM((ncmax, CH), jnp.int32),
            pltpu.VMEM((CH, 16), jnp.float32),
            pltpu.VMEM_SHARED((nacc, 16), jnp.float32),
        ],
        compiler_params=pltpu.CompilerParams(use_tc_tiling_on_sc=False),
    )
    degp = deg_call(dst, ones16, zeros16)
    d0, d1 = degp[0], degp[1]

    agg_call = pl.kernel(
        functools.partial(_agg_kernel, nc0, nc1, nacc),
        out_type=jax.ShapeDtypeStruct((NC, nacc, h), jnp.float32),
        mesh=_sc_mesh(),
        scratch_types=[
            pltpu.VMEM((ncmax, CH), jnp.int32),
            pltpu.VMEM((ncmax, CH), jnp.int32),
        ] + [pltpu.VMEM((CH, h), jnp.float32)] * NBUF + [
            pltpu.VMEM_SHARED((nacc, h), jnp.float32),
        ] + [pltpu.SemaphoreType.DMA] * (2 * NBUF),
        compiler_params=pltpu.CompilerParams(use_tc_tiling_on_sc=False),
    )

    h1p = pl.pallas_call(
        functools.partial(_stage_a_body, n),
        out_shape=jax.ShapeDtypeStruct((n, h), jnp.float32),
    )(x, W1, d0, d1)

    agg1 = agg_call(h1p, src, dst, zerosh)

    h2p = pl.pallas_call(
        functools.partial(_stage_b_body, n),
        out_shape=jax.ShapeDtypeStruct((n, h), jnp.float32),
    )(agg1[0], agg1[1], h1p, d0, d1, b1.reshape(1, h), W2)

    agg2 = agg_call(h2p, src, dst, zerosh)

    out = pl.pallas_call(
        functools.partial(_stage_c_body, n, g),
        out_shape=jax.ShapeDtypeStruct((g, 1), jnp.float32),
        in_specs=[pl.BlockSpec(memory_space=pltpu.SMEM)] +
                 [pl.BlockSpec()] * 8,
    )(ptr, agg2[0], agg2[1], h2p, d0, d1, b2.reshape(1, h), Wp,
      bp.reshape(1, 1))
    return out


# speedup vs baseline: 3.9177x; 3.9177x over previous
import functools

import jax
import jax.numpy as jnp
import numpy as np
from jax import lax
from jax.experimental import pallas as pl
from jax.experimental.pallas import tpu as pltpu
from jax.experimental.pallas import tpu_sc as plsc

NC = 2
NS = 16
NW = NC * NS
CH = 128
NBUF = 8


def _sc_mesh():
    return plsc.VectorSubcoreMesh(core_axis_name="c", subcore_axis_name="s")


def _deg_kernel(nc0, nc1, nacc, dst_hbm, ones_hbm, zeros_hbm, out_hbm,
                dst_v, ones_v, accum):
    cid = lax.axis_index("c")
    sid = lax.axis_index("s")
    nchunk = lax.select(cid == 0, jnp.int32(nc0), jnp.int32(nc1))
    rows = nacc // NS
    pltpu.sync_copy(dst_hbm.at[cid, sid], dst_v)
    pltpu.sync_copy(ones_hbm, ones_v)
    pltpu.sync_copy(zeros_hbm, accum.at[pl.ds(sid * rows, rows)])
    plsc.subcore_barrier()

    def body(j, carry):
        pltpu.sync_copy(ones_v, accum.at[dst_v.at[j]], add=True)
        return carry

    lax.fori_loop(0, nchunk, body, 0)
    plsc.subcore_barrier()
    pltpu.sync_copy(accum.at[pl.ds(sid * rows, rows)],
                    out_hbm.at[cid, pl.ds(sid * rows, rows)])


def _agg_kernel(nc0, nc1, nacc, hp_hbm, src_hbm, dst_hbm, zeros_hbm, out_hbm,
                src_v, dst_v, *scratch):
    cid = lax.axis_index("c")
    sid = lax.axis_index("s")
    nchunk = lax.select(cid == 0, jnp.int32(nc0), jnp.int32(nc1))
    rows = nacc // NS
    bufs = scratch[:NBUF]
    accum = scratch[NBUF]
    gsem = scratch[NBUF + 1:2 * NBUF + 1]
    ssem = scratch[2 * NBUF + 1:]
    pltpu.sync_copy(src_hbm.at[cid, sid], src_v)
    pltpu.sync_copy(dst_hbm.at[cid, sid], dst_v)
    pltpu.sync_copy(zeros_hbm, accum.at[pl.ds(sid * rows, rows)])
    plsc.subcore_barrier()

    for b in range(NBUF):
        pltpu.async_copy(hp_hbm.at[src_v.at[b]], bufs[b], gsem[b])

    def body(i, carry):
        base = i * NBUF
        for b in range(NBUF):
            j = base + b
            pltpu.make_async_copy(hp_hbm.at[src_v.at[j]], bufs[b],
                                  gsem[b]).wait()
            pltpu.async_copy(bufs[b], accum.at[dst_v.at[j]], ssem[b],
                             add=True)
        for b in range(NBUF):
            j = base + b
            jn = j + NBUF
            pltpu.make_async_copy(bufs[b], accum.at[dst_v.at[j]],
                                  ssem[b]).wait()

            @pl.when(jn < nchunk)
            def _():
                pltpu.async_copy(hp_hbm.at[src_v.at[jn]], bufs[b], gsem[b])

        return carry

    lax.fori_loop(0, nchunk // NBUF, body, 0)
    plsc.subcore_barrier()
    pltpu.sync_copy(accum.at[pl.ds(sid * rows, rows)],
                    out_hbm.at[cid, pl.ds(sid * rows, rows)])


def _stage_a_body(n, x_ref, w1_ref, d_ref, o_ref):
    deg = d_ref[0, :n, 0:1] + d_ref[1, :n, 0:1] + 1.0
    dis = lax.rsqrt(deg)
    hw = jnp.dot(x_ref[...], w1_ref[...], preferred_element_type=jnp.float32)
    o_ref[...] = hw * dis


def _stage_b_body(n, a_ref, hp_ref, d_ref, b1_ref, w2_ref, o_ref):
    deg = d_ref[0, :n, 0:1] + d_ref[1, :n, 0:1] + 1.0
    dis = lax.rsqrt(deg)
    agg = a_ref[0, :n] + a_ref[1, :n] + hp_ref[...]
    h1 = jax.nn.relu(agg * dis + b1_ref[...])
    o_ref[...] = jnp.dot(h1, w2_ref[...], preferred_element_type=jnp.float32) * dis


def _stage_c_body(n, g, ptr_ref, a_ref, hp_ref, d_ref, b2_ref, wp_ref,
                  bp_ref, o_ref):
    deg = d_ref[0, :n, 0:1] + d_ref[1, :n, 0:1] + 1.0
    dis = lax.rsqrt(deg)
    agg = a_ref[0, :n] + a_ref[1, :n] + hp_ref[...]
    h2 = jax.nn.relu(agg * dis + b2_ref[...])
    idx = lax.broadcasted_iota(jnp.int32, (n, 1), 0)
    means = []
    for gi in range(g):
        lo = ptr_ref[gi]
        hi = ptr_ref[gi + 1]
        m = (idx >= lo) & (idx < hi)
        s = jnp.sum(jnp.where(m, h2, 0.0), axis=0, keepdims=True)
        cnt = jnp.maximum((hi - lo).astype(jnp.float32), 1.0)
        means.append(s / cnt)
    mean = jnp.concatenate(means, axis=0)
    o_ref[...] = jnp.dot(mean, wp_ref[...],
                         preferred_element_type=jnp.float32) + bp_ref[...]


def kernel(x, edge_index, ptr, W1, b1, W2, b2, Wp, bp):
    n, d = x.shape
    h = W1.shape[1]
    g = ptr.shape[0] - 1
    e = edge_index.shape[1]

    F0 = 0.5
    ept0 = max(int(e * F0 / NS) // (CH * NBUF) * (CH * NBUF), CH * NBUF)
    e1 = max(e - NS * ept0, 0)
    ept1 = max(-(-(-(-e1 // NS)) // (CH * NBUF)) * (CH * NBUF), CH * NBUF)
    nc0, nc1 = ept0 // CH, ept1 // CH
    ncmax = max(nc0, nc1)
    rows_per_tile = ((-(-n // NS)) + 7) // 8 * 8
    nacc = NS * rows_per_tile

    def _layout(flat, fill):
        b0 = flat[:NS * ept0].reshape(NS, nc0, CH)
        b0 = jnp.pad(b0, ((0, 0), (0, ncmax - nc0), (0, 0)),
                     constant_values=fill)
        b1 = jnp.pad(flat[NS * ept0:], (0, NS * ept1 - (e - NS * ept0)),
                     constant_values=fill).reshape(NS, nc1, CH)
        b1 = jnp.pad(b1, ((0, 0), (0, ncmax - nc1), (0, 0)),
                     constant_values=fill)
        return jnp.stack([b0, b1])

    src = _layout(edge_index[0], 0)
    dst = _layout(edge_index[1], n)

    ones16 = jnp.ones((CH, 16), jnp.float32)
    zeros16 = jnp.zeros((rows_per_tile, 16), jnp.float32)
    zerosh = jnp.zeros((rows_per_tile, h), jnp.float32)

    deg_call = pl.kernel(
        functools.partial(_deg_kernel, nc0, nc1, nacc),
        out_type=jax.ShapeDtypeStruct((NC, nacc, 16), jnp.float32),
        mesh=_sc_mesh(),
        scratch_types=[
            pltpu.VMEM((ncmax, CH), jnp.int32),
            pltpu.VMEM((CH, 16), jnp.float32),
            pltpu.VMEM_SHARED((nacc, 16), jnp.float32),
        ],
        compiler_params=pltpu.CompilerParams(use_tc_tiling_on_sc=False),
    )
    degp = deg_call(dst, ones16, zeros16)

    agg_call = pl.kernel(
        functools.partial(_agg_kernel, nc0, nc1, nacc),
        out_type=jax.ShapeDtypeStruct((NC, nacc, h), jnp.float32),
        mesh=_sc_mesh(),
        scratch_types=[
            pltpu.VMEM((ncmax, CH), jnp.int32),
            pltpu.VMEM((ncmax, CH), jnp.int32),
        ] + [pltpu.VMEM((CH, h), jnp.float32)] * NBUF + [
            pltpu.VMEM_SHARED((nacc, h), jnp.float32),
        ] + [pltpu.SemaphoreType.DMA] * (2 * NBUF),
        compiler_params=pltpu.CompilerParams(use_tc_tiling_on_sc=False),
    )

    h1p = pl.pallas_call(
        functools.partial(_stage_a_body, n),
        out_shape=jax.ShapeDtypeStruct((n, h), jnp.float32),
    )(x, W1, degp)

    agg1 = agg_call(h1p, src, dst, zerosh)

    h2p = pl.pallas_call(
        functools.partial(_stage_b_body, n),
        out_shape=jax.ShapeDtypeStruct((n, h), jnp.float32),
    )(agg1, h1p, degp, b1.reshape(1, h), W2)

    agg2 = agg_call(h2p, src, dst, zerosh)

    out = pl.pallas_call(
        functools.partial(_stage_c_body, n, g),
        out_shape=jax.ShapeDtypeStruct((g, 1), jnp.float32),
        in_specs=[pl.BlockSpec(memory_space=pltpu.SMEM)] +
                 [pl.BlockSpec()] * 6,
    )(ptr, agg2, h2p, degp, b2.reshape(1, h), Wp, bp.reshape(1, 1))
    return out


# speedup vs baseline: 3.9203x; 1.0007x over previous
import functools

import jax
import jax.numpy as jnp
import numpy as np
from jax import lax
from jax.experimental import pallas as pl
from jax.experimental.pallas import tpu as pltpu
from jax.experimental.pallas import tpu_sc as plsc

NC = 2
NS = 16
NW = NC * NS
CH = 128
NBUF = 8


def _sc_mesh():
    return plsc.VectorSubcoreMesh(core_axis_name="c", subcore_axis_name="s")


def _deg_kernel(nc0, nc1, nacc, dst_hbm, ones_hbm, zeros_hbm, out_hbm,
                dst_v, ones_v, accum):
    cid = lax.axis_index("c")
    sid = lax.axis_index("s")
    nchunk = lax.select(cid == 0, jnp.int32(nc0), jnp.int32(nc1))
    rows = nacc // NS
    pltpu.sync_copy(dst_hbm.at[cid, sid], dst_v)
    pltpu.sync_copy(ones_hbm, ones_v)
    pltpu.sync_copy(zeros_hbm, accum.at[pl.ds(sid * rows, rows)])
    plsc.subcore_barrier()

    def body(j, carry):
        pltpu.sync_copy(ones_v, accum.at[dst_v.at[j]], add=True)
        return carry

    lax.fori_loop(0, nchunk, body, 0)
    plsc.subcore_barrier()
    pltpu.sync_copy(accum.at[pl.ds(sid * rows, rows)],
                    out_hbm.at[cid, pl.ds(sid * rows, rows)])


def _agg_kernel(nc0, nc1, nacc, hp_hbm, src_hbm, dst_hbm, zeros_hbm, out_hbm,
                src_v, dst_v, *scratch):
    cid = lax.axis_index("c")
    sid = lax.axis_index("s")
    nchunk = lax.select(cid == 0, jnp.int32(nc0), jnp.int32(nc1))
    rows = nacc // NS
    bufs = scratch[:NBUF]
    accum = scratch[NBUF]
    gsem = scratch[NBUF + 1:2 * NBUF + 1]
    ssem = scratch[2 * NBUF + 1:]
    pltpu.sync_copy(src_hbm.at[cid, sid], src_v)
    pltpu.sync_copy(dst_hbm.at[cid, sid], dst_v)
    pltpu.sync_copy(zeros_hbm, accum.at[pl.ds(sid * rows, rows)])
    plsc.subcore_barrier()

    for b in range(NBUF):
        pltpu.async_copy(hp_hbm.at[src_v.at[b]], bufs[b], gsem[b])

    def body(i, carry):
        base = i * NBUF
        for b in range(NBUF):
            j = base + b
            pltpu.make_async_copy(hp_hbm.at[src_v.at[j]], bufs[b],
                                  gsem[b]).wait()
            pltpu.async_copy(bufs[b], accum.at[dst_v.at[j]], ssem[b],
                             add=True)
        for b in range(NBUF):
            j = base + b
            jn = j + NBUF
            pltpu.make_async_copy(bufs[b], accum.at[dst_v.at[j]],
                                  ssem[b]).wait()

            @pl.when(jn < nchunk)
            def _():
                pltpu.async_copy(hp_hbm.at[src_v.at[jn]], bufs[b], gsem[b])

        return carry

    lax.fori_loop(0, nchunk // NBUF, body, 0)
    plsc.subcore_barrier()
    pltpu.sync_copy(accum.at[pl.ds(sid * rows, rows)],
                    out_hbm.at[cid, pl.ds(sid * rows, rows)])


def _stage_a_body(n, x_ref, w1_ref, d_ref, o_ref):
    deg = d_ref[0, :n, 0:1] + d_ref[1, :n, 0:1] + 1.0
    dis = lax.rsqrt(deg)
    hw = jnp.dot(x_ref[...], w1_ref[...], preferred_element_type=jnp.float32)
    o_ref[...] = hw * dis


def _stage_b_body(n, a_ref, hp_ref, d_ref, b1_ref, w2_ref, o_ref):
    deg = d_ref[0, :n, 0:1] + d_ref[1, :n, 0:1] + 1.0
    dis = lax.rsqrt(deg)
    agg = a_ref[0, :n] + a_ref[1, :n] + hp_ref[...]
    h1 = jax.nn.relu(agg * dis + b1_ref[...])
    o_ref[...] = jnp.dot(h1, w2_ref[...], preferred_element_type=jnp.float32) * dis


def _stage_c_body(n, g, ptr_ref, a_ref, hp_ref, d_ref, b2_ref, wp_ref,
                  bp_ref, o_ref):
    deg = d_ref[0, :n, 0:1] + d_ref[1, :n, 0:1] + 1.0
    dis = lax.rsqrt(deg)
    agg = a_ref[0, :n] + a_ref[1, :n] + hp_ref[...]
    h2 = jax.nn.relu(agg * dis + b2_ref[...])
    idx = lax.broadcasted_iota(jnp.int32, (n, 1), 0)
    means = []
    for gi in range(g):
        lo = ptr_ref[gi]
        hi = ptr_ref[gi + 1]
        m = (idx >= lo) & (idx < hi)
        s = jnp.sum(jnp.where(m, h2, 0.0), axis=0, keepdims=True)
        cnt = jnp.maximum((hi - lo).astype(jnp.float32), 1.0)
        means.append(s / cnt)
    mean = jnp.concatenate(means, axis=0)
    o_ref[...] = jnp.dot(mean, wp_ref[...],
                         preferred_element_type=jnp.float32) + bp_ref[...]


def kernel(x, edge_index, ptr, W1, b1, W2, b2, Wp, bp):
    n, d = x.shape
    h = W1.shape[1]
    g = ptr.shape[0] - 1
    e = edge_index.shape[1]

    F0 = 0.5
    ept0 = max(int(e * F0 / NS) // (CH * NBUF) * (CH * NBUF), CH * NBUF)
    e1 = max(e - NS * ept0, 0)
    ept1 = max(-(-(-(-e1 // NS)) // (CH * NBUF)) * (CH * NBUF), CH * NBUF)
    nc0, nc1 = ept0 // CH, ept1 // CH
    ncmax = max(nc0, nc1)
    rows_per_tile = ((-(-n // NS)) + 7) // 8 * 8
    nacc = NS * rows_per_tile

    def _layout(flat, fill):
        if nc0 == nc1:
            pad = jnp.full((NC * NS * ncmax * CH - e,), fill, jnp.int32)
            return jnp.concatenate([flat, pad]).reshape(NC, NS, ncmax, CH)
        b0 = flat[:NS * ept0].reshape(NS, nc0, CH)
        b0 = jnp.pad(b0, ((0, 0), (0, ncmax - nc0), (0, 0)),
                     constant_values=fill)
        b1 = jnp.pad(flat[NS * ept0:], (0, NS * ept1 - (e - NS * ept0)),
                     constant_values=fill).reshape(NS, nc1, CH)
        b1 = jnp.pad(b1, ((0, 0), (0, ncmax - nc1), (0, 0)),
                     constant_values=fill)
        return jnp.stack([b0, b1])

    src = _layout(edge_index[0], 0)
    dst = _layout(edge_index[1], n)

    ones16 = jnp.ones((CH, 16), jnp.float32)
    zeros16 = jnp.zeros((rows_per_tile, 16), jnp.float32)
    zerosh = jnp.zeros((rows_per_tile, h), jnp.float32)

    deg_call = pl.kernel(
        functools.partial(_deg_kernel, nc0, nc1, nacc),
        out_type=jax.ShapeDtypeStruct((NC, nacc, 16), jnp.float32),
        mesh=_sc_mesh(),
        scratch_types=[
            pltpu.VMEM((ncmax, CH), jnp.int32),
            pltpu.VMEM((CH, 16), jnp.float32),
            pltpu.VMEM_SHARED((nacc, 16), jnp.float32),
        ],
        compiler_params=pltpu.CompilerParams(use_tc_tiling_on_sc=False),
    )
    degp = deg_call(dst, ones16, zeros16)

    agg_call = pl.kernel(
        functools.partial(_agg_kernel, nc0, nc1, nacc),
        out_type=jax.ShapeDtypeStruct((NC, nacc, h), jnp.float32),
        mesh=_sc_mesh(),
        scratch_types=[
            pltpu.VMEM((ncmax, CH), jnp.int32),
            pltpu.VMEM((ncmax, CH), jnp.int32),
        ] + [pltpu.VMEM((CH, h), jnp.float32)] * NBUF + [
            pltpu.VMEM_SHARED((nacc, h), jnp.float32),
        ] + [pltpu.SemaphoreType.DMA] * (2 * NBUF),
        compiler_params=pltpu.CompilerParams(use_tc_tiling_on_sc=False),
    )

    h1p = pl.pallas_call(
        functools.partial(_stage_a_body, n),
        out_shape=jax.ShapeDtypeStruct((n, h), jnp.float32),
    )(x, W1, degp)

    agg1 = agg_call(h1p, src, dst, zerosh)

    h2p = pl.pallas_call(
        functools.partial(_stage_b_body, n),
        out_shape=jax.ShapeDtypeStruct((n, h), jnp.float32),
    )(agg1, h1p, degp, b1.reshape(1, h), W2)

    agg2 = agg_call(h2p, src, dst, zerosh)

    out = pl.pallas_call(
        functools.partial(_stage_c_body, n, g),
        out_shape=jax.ShapeDtypeStruct((g, 1), jnp.float32),
        in_specs=[pl.BlockSpec(memory_space=pltpu.SMEM)] +
                 [pl.BlockSpec()] * 6,
    )(ptr, agg2, h2p, degp, b2.reshape(1, h), Wp, bp.reshape(1, 1))
    return out


# speedup vs baseline: 4.7286x; 1.2062x over previous
import functools

import jax
import jax.numpy as jnp
import numpy as np
from jax import lax
from jax.experimental import pallas as pl
from jax.experimental.pallas import tpu as pltpu
from jax.experimental.pallas import tpu_sc as plsc

NC = 2
NS = 16
NW = NC * NS
CH = 128
NBUF = 8


def _sc_mesh():
    return plsc.VectorSubcoreMesh(core_axis_name="c", subcore_axis_name="s")


def _deg_kernel(nc0, nc1, nacc, dst_hbm, ones_hbm, zeros_hbm, out_hbm,
                dst_v, ones_v, accum):
    cid = lax.axis_index("c")
    sid = lax.axis_index("s")
    nchunk = lax.select(cid == 0, jnp.int32(nc0), jnp.int32(nc1))
    rows = nacc // NS
    pltpu.sync_copy(dst_hbm.at[cid, sid], dst_v)
    pltpu.sync_copy(ones_hbm, ones_v)
    pltpu.sync_copy(zeros_hbm, accum.at[pl.ds(sid * rows, rows)])
    plsc.subcore_barrier()

    def body(j, carry):
        pltpu.sync_copy(ones_v, accum.at[dst_v.at[j]], add=True)
        return carry

    lax.fori_loop(0, nchunk, body, 0)
    plsc.subcore_barrier()
    pltpu.sync_copy(accum.at[pl.ds(sid * rows, rows)],
                    out_hbm.at[cid, pl.ds(sid * rows, rows)])


def _agg_kernel(nc0, nc1, nacc, hp_hbm, src_hbm, dst_hbm, zeros_hbm, out_hbm,
                src_v, dst_v, *scratch):
    cid = lax.axis_index("c")
    sid = lax.axis_index("s")
    nchunk = lax.select(cid == 0, jnp.int32(nc0), jnp.int32(nc1))
    rows = nacc // NS
    bufs = scratch[:NBUF]
    accum = scratch[NBUF]
    gsem = scratch[NBUF + 1:2 * NBUF + 1]
    ssem = scratch[2 * NBUF + 1:]
    pltpu.sync_copy(src_hbm.at[cid, sid], src_v)
    pltpu.sync_copy(dst_hbm.at[cid, sid], dst_v)
    pltpu.sync_copy(zeros_hbm, accum.at[pl.ds(sid * rows, rows)])
    plsc.subcore_barrier()

    for b in range(NBUF):
        pltpu.async_copy(hp_hbm.at[src_v.at[b]], bufs[b], gsem[b])

    def body(i, carry):
        base = i * NBUF
        for b in range(NBUF):
            j = base + b
            pltpu.make_async_copy(hp_hbm.at[src_v.at[j]], bufs[b],
                                  gsem[b]).wait()
            pltpu.async_copy(bufs[b], accum.at[dst_v.at[j]], ssem[b],
                             add=True)
        for b in range(NBUF):
            j = base + b
            jn = j + NBUF
            pltpu.make_async_copy(bufs[b], accum.at[dst_v.at[j]],
                                  ssem[b]).wait()

            @pl.when(jn < nchunk)
            def _():
                pltpu.async_copy(hp_hbm.at[src_v.at[jn]], bufs[b], gsem[b])

        return carry

    lax.fori_loop(0, nchunk // NBUF, body, 0)
    plsc.subcore_barrier()
    pltpu.sync_copy(accum.at[pl.ds(sid * rows, rows)],
                    out_hbm.at[cid, pl.ds(sid * rows, rows)])


def _stage_a_body(n, x_ref, w1_ref, d_ref, o_ref):
    deg = d_ref[0, :n, 0:1] + d_ref[1, :n, 0:1] + 1.0
    dis = lax.rsqrt(deg)
    hw = jnp.dot(x_ref[...], w1_ref[...], preferred_element_type=jnp.float32)
    o_ref[...] = hw * dis


def _stage_b_body(n, a_ref, hp_ref, d_ref, b1_ref, w2_ref, o_ref):
    deg = d_ref[0, :n, 0:1] + d_ref[1, :n, 0:1] + 1.0
    dis = lax.rsqrt(deg)
    agg = a_ref[0, :n] + a_ref[1, :n] + hp_ref[...]
    h1 = jax.nn.relu(agg * dis + b1_ref[...])
    o_ref[...] = jnp.dot(h1, w2_ref[...], preferred_element_type=jnp.float32) * dis


def _stage_c_body(n, g, ptr_ref, a_ref, hp_ref, d_ref, b2_ref, wp_ref,
                  bp_ref, o_ref):
    deg = d_ref[0, :n, 0:1] + d_ref[1, :n, 0:1] + 1.0
    dis = lax.rsqrt(deg)
    agg = a_ref[0, :n] + a_ref[1, :n] + hp_ref[...]
    h2 = jax.nn.relu(agg * dis + b2_ref[...])
    idx = lax.broadcasted_iota(jnp.int32, (n, 1), 0)
    means = []
    for gi in range(g):
        lo = ptr_ref[gi]
        hi = ptr_ref[gi + 1]
        m = (idx >= lo) & (idx < hi)
        s = jnp.sum(jnp.where(m, h2, 0.0), axis=0, keepdims=True)
        cnt = jnp.maximum((hi - lo).astype(jnp.float32), 1.0)
        means.append(s / cnt)
    mean = jnp.concatenate(means, axis=0)
    o_ref[...] = jnp.dot(mean, wp_ref[...],
                         preferred_element_type=jnp.float32) + bp_ref[...]


def kernel(x, edge_index, ptr, W1, b1, W2, b2, Wp, bp):
    n, d = x.shape
    h = W1.shape[1]
    g = ptr.shape[0] - 1
    e = edge_index.shape[1]

    F0 = 0.875
    ept0 = max(int(e * F0 / NS) // (CH * NBUF) * (CH * NBUF), CH * NBUF)
    e1 = max(e - NS * ept0, 0)
    ept1 = max(-(-(-(-e1 // NS)) // (CH * NBUF)) * (CH * NBUF), CH * NBUF)
    nc0, nc1 = ept0 // CH, ept1 // CH
    ncmax = max(nc0, nc1)
    rows_per_tile = ((-(-n // NS)) + 7) // 8 * 8
    nacc = NS * rows_per_tile

    def _layout(flat, fill):
        if nc0 == nc1:
            pad = jnp.full((NC * NS * ncmax * CH - e,), fill, jnp.int32)
            return jnp.concatenate([flat, pad]).reshape(NC, NS, ncmax, CH)
        b0 = flat[:NS * ept0].reshape(NS, nc0, CH)
        b0 = jnp.pad(b0, ((0, 0), (0, ncmax - nc0), (0, 0)),
                     constant_values=fill)
        b1 = jnp.pad(flat[NS * ept0:], (0, NS * ept1 - (e - NS * ept0)),
                     constant_values=fill).reshape(NS, nc1, CH)
        b1 = jnp.pad(b1, ((0, 0), (0, ncmax - nc1), (0, 0)),
                     constant_values=fill)
        return jnp.stack([b0, b1])

    src = _layout(edge_index[0], 0)
    dst = _layout(edge_index[1], n)

    ones16 = jnp.ones((CH, 16), jnp.float32)
    zeros16 = jnp.zeros((rows_per_tile, 16), jnp.float32)
    zerosh = jnp.zeros((rows_per_tile, h), jnp.float32)

    deg_call = pl.kernel(
        functools.partial(_deg_kernel, nc0, nc1, nacc),
        out_type=jax.ShapeDtypeStruct((NC, nacc, 16), jnp.float32),
        mesh=_sc_mesh(),
        scratch_types=[
            pltpu.VMEM((ncmax, CH), jnp.int32),
            pltpu.VMEM((CH, 16), jnp.float32),
            pltpu.VMEM_SHARED((nacc, 16), jnp.float32),
        ],
        compiler_params=pltpu.CompilerParams(use_tc_tiling_on_sc=False),
    )
    degp = deg_call(dst, ones16, zeros16)

    agg_call = pl.kernel(
        functools.partial(_agg_kernel, nc0, nc1, nacc),
        out_type=jax.ShapeDtypeStruct((NC, nacc, h), jnp.float32),
        mesh=_sc_mesh(),
        scratch_types=[
            pltpu.VMEM((ncmax, CH), jnp.int32),
            pltpu.VMEM((ncmax, CH), jnp.int32),
        ] + [pltpu.VMEM((CH, h), jnp.float32)] * NBUF + [
            pltpu.VMEM_SHARED((nacc, h), jnp.float32),
        ] + [pltpu.SemaphoreType.DMA] * (2 * NBUF),
        compiler_params=pltpu.CompilerParams(use_tc_tiling_on_sc=False),
    )

    h1p = pl.pallas_call(
        functools.partial(_stage_a_body, n),
        out_shape=jax.ShapeDtypeStruct((n, h), jnp.float32),
    )(x, W1, degp)

    agg1 = agg_call(h1p, src, dst, zerosh)

    h2p = pl.pallas_call(
        functools.partial(_stage_b_body, n),
        out_shape=jax.ShapeDtypeStruct((n, h), jnp.float32),
    )(agg1, h1p, degp, b1.reshape(1, h), W2)

    agg2 = agg_call(h2p, src, dst, zerosh)

    out = pl.pallas_call(
        functools.partial(_stage_c_body, n, g),
        out_shape=jax.ShapeDtypeStruct((g, 1), jnp.float32),
        in_specs=[pl.BlockSpec(memory_space=pltpu.SMEM)] +
                 [pl.BlockSpec()] * 6,
    )(ptr, agg2, h2p, degp, b2.reshape(1, h), Wp, bp.reshape(1, 1))
    return out


# speedup vs baseline: 4.7356x; 1.0015x over previous
import functools

import jax
import jax.numpy as jnp
from jax import lax
from jax.experimental import pallas as pl
from jax.experimental.pallas import tpu as pltpu
from jax.experimental.pallas import tpu_sc as plsc

NC = 2
NS = 16
NW = NC * NS
CH = 128
NBUF = 8


def _sc_mesh():
    return plsc.VectorSubcoreMesh(core_axis_name="c", subcore_axis_name="s")


def _deg_kernel(nc0, nc1, nacc, dst_hbm, ones_hbm, zeros_hbm, out_hbm,
                dst_v, ones_v, accum):
    cid = lax.axis_index("c")
    sid = lax.axis_index("s")
    nchunk = lax.select(cid == 0, jnp.int32(nc0), jnp.int32(nc1))
    rows = nacc // NS
    pltpu.sync_copy(dst_hbm.at[cid, sid], dst_v)
    pltpu.sync_copy(ones_hbm, ones_v)
    pltpu.sync_copy(zeros_hbm, accum.at[pl.ds(sid * rows, rows)])
    plsc.subcore_barrier()

    def body(j, carry):
        pltpu.sync_copy(ones_v, accum.at[dst_v.at[j]], add=True)
        return carry

    lax.fori_loop(0, nchunk, body, 0)
    plsc.subcore_barrier()
    pltpu.sync_copy(accum.at[pl.ds(sid * rows, rows)],
                    out_hbm.at[cid, pl.ds(sid * rows, rows)])


def _agg_kernel(nc0, nc1, nacc, hp_hbm, src_hbm, dst_hbm, zeros_hbm, out_hbm,
                src_v, dst_v, *scratch):
    cid = lax.axis_index("c")
    sid = lax.axis_index("s")
    nchunk = lax.select(cid == 0, jnp.int32(nc0), jnp.int32(nc1))
    rows = nacc // NS
    bufs = scratch[:NBUF]
    accum = scratch[NBUF]
    gsem = scratch[NBUF + 1:2 * NBUF + 1]
    ssem = scratch[2 * NBUF + 1:]
    pltpu.sync_copy(src_hbm.at[cid, sid], src_v)
    pltpu.sync_copy(dst_hbm.at[cid, sid], dst_v)
    pltpu.sync_copy(zeros_hbm, accum.at[pl.ds(sid * rows, rows)])
    plsc.subcore_barrier()

    for b in range(NBUF):
        pltpu.async_copy(hp_hbm.at[src_v.at[b]], bufs[b], gsem[b])

    def body(i, carry):
        base = i * NBUF
        for b in range(NBUF):
            j = base + b
            pltpu.make_async_copy(hp_hbm.at[src_v.at[j]], bufs[b],
                                  gsem[b]).wait()
            pltpu.async_copy(bufs[b], accum.at[dst_v.at[j]], ssem[b],
                             add=True)
        for b in range(NBUF):
            j = base + b
            jn = j + NBUF
            pltpu.make_async_copy(bufs[b], accum.at[dst_v.at[j]],
                                  ssem[b]).wait()

            @pl.when(jn < nchunk)
            def _():
                pltpu.async_copy(hp_hbm.at[src_v.at[jn]], bufs[b], gsem[b])

        return carry

    lax.fori_loop(0, nchunk // NBUF, body, 0)
    plsc.subcore_barrier()
    pltpu.sync_copy(accum.at[pl.ds(sid * rows, rows)],
                    out_hbm.at[cid, pl.ds(sid * rows, rows)])


def _stage_a_body(n, x_ref, w1_ref, d_ref, o_ref):
    deg = d_ref[0, :n, 0:1] + d_ref[1, :n, 0:1] + 1.0
    dis = lax.rsqrt(deg)
    hw = jnp.dot(x_ref[...], w1_ref[...], preferred_element_type=jnp.float32)
    o_ref[...] = hw * dis


def _stage_b_body(n, a_ref, hp_ref, d_ref, b1_ref, w2_ref, o_ref):
    deg = d_ref[0, :n, 0:1] + d_ref[1, :n, 0:1] + 1.0
    dis = lax.rsqrt(deg)
    agg = a_ref[0, :n] + a_ref[1, :n] + hp_ref[...]
    h1 = jax.nn.relu(agg * dis + b1_ref[...])
    o_ref[...] = jnp.dot(h1, w2_ref[...], preferred_element_type=jnp.float32) * dis


def _stage_c_body(n, g, ptr_ref, a_ref, hp_ref, d_ref, b2_ref, wp_ref,
                  bp_ref, o_ref):
    deg = d_ref[0, :n, 0:1] + d_ref[1, :n, 0:1] + 1.0
    dis = lax.rsqrt(deg)
    agg = a_ref[0, :n] + a_ref[1, :n] + hp_ref[...]
    h2 = jax.nn.relu(agg * dis + b2_ref[...])
    idx = lax.broadcasted_iota(jnp.int32, (n, 1), 0)
    means = []
    for gi in range(g):
        lo = ptr_ref[gi]
        hi = ptr_ref[gi + 1]
        m = (idx >= lo) & (idx < hi)
        s = jnp.sum(jnp.where(m, h2, 0.0), axis=0, keepdims=True)
        cnt = jnp.maximum((hi - lo).astype(jnp.float32), 1.0)
        means.append(s / cnt)
    mean = jnp.concatenate(means, axis=0)
    o_ref[...] = jnp.dot(mean, wp_ref[...],
                         preferred_element_type=jnp.float32) + bp_ref[...]


def kernel(x, edge_index, ptr, W1, b1, W2, b2, Wp, bp):
    n, d = x.shape
    h = W1.shape[1]
    g = ptr.shape[0] - 1
    e = edge_index.shape[1]

    F0 = 0.875
    ept0 = max(int(e * F0 / NS) // (CH * NBUF) * (CH * NBUF), CH * NBUF)
    e1 = max(e - NS * ept0, 0)
    ept1 = max(-(-(-(-e1 // NS)) // (CH * NBUF)) * (CH * NBUF), CH * NBUF)
    nc0, nc1 = ept0 // CH, ept1 // CH
    ncmax = max(nc0, nc1)
    rows_per_tile = ((-(-n // NS)) + 7) // 8 * 8
    nacc = NS * rows_per_tile

    def _layout(flat, fill):
        if nc0 == nc1:
            pad = jnp.full((NC * NS * ncmax * CH - e,), fill, jnp.int32)
            return jnp.concatenate([flat, pad]).reshape(NC, NS, ncmax, CH)
        b0 = flat[:NS * ept0].reshape(NS, nc0, CH)
        b0 = jnp.pad(b0, ((0, 0), (0, ncmax - nc0), (0, 0)),
                     constant_values=fill)
        b1 = jnp.pad(flat[NS * ept0:], (0, NS * ept1 - (e - NS * ept0)),
                     constant_values=fill).reshape(NS, nc1, CH)
        b1 = jnp.pad(b1, ((0, 0), (0, ncmax - nc1), (0, 0)),
                     constant_values=fill)
        return jnp.stack([b0, b1])

    src = _layout(edge_index[0], 0)
    dst = _layout(edge_index[1], n)

    ones16 = jnp.ones((CH, 16), jnp.float32)
    zeros16 = jnp.zeros((rows_per_tile, 16), jnp.float32)
    zerosh = jnp.zeros((rows_per_tile, h), jnp.float32)

    deg_call = pl.kernel(
        functools.partial(_deg_kernel, nc0, nc1, nacc),
        out_type=jax.ShapeDtypeStruct((NC, nacc, 16), jnp.float32),
        mesh=_sc_mesh(),
        scratch_types=[
            pltpu.VMEM((ncmax, CH), jnp.int32),
            pltpu.VMEM((CH, 16), jnp.float32),
            pltpu.VMEM_SHARED((nacc, 16), jnp.float32),
        ],
        compiler_params=pltpu.CompilerParams(use_tc_tiling_on_sc=False),
    )
    degp = deg_call(dst, ones16, zeros16)

    agg_call = pl.kernel(
        functools.partial(_agg_kernel, nc0, nc1, nacc),
        out_type=jax.ShapeDtypeStruct((NC, nacc, h), jnp.float32),
        mesh=_sc_mesh(),
        scratch_types=[
            pltpu.VMEM((ncmax, CH), jnp.int32),
            pltpu.VMEM((ncmax, CH), jnp.int32),
        ] + [pltpu.VMEM((CH, h), jnp.float32)] * NBUF + [
            pltpu.VMEM_SHARED((nacc, h), jnp.float32),
        ] + [pltpu.SemaphoreType.DMA] * (2 * NBUF),
        compiler_params=pltpu.CompilerParams(use_tc_tiling_on_sc=False),
    )

    h1p = pl.pallas_call(
        functools.partial(_stage_a_body, n),
        out_shape=jax.ShapeDtypeStruct((n, h), jnp.float32),
    )(x, W1, degp)

    agg1 = agg_call(h1p, src, dst, zerosh)

    h2p = pl.pallas_call(
        functools.partial(_stage_b_body, n),
        out_shape=jax.ShapeDtypeStruct((n, h), jnp.float32),
    )(agg1, h1p, degp, b1.reshape(1, h), W2)

    agg2 = agg_call(h2p, src, dst, zerosh)

    out = pl.pallas_call(
        functools.partial(_stage_c_body, n, g),
        out_shape=jax.ShapeDtypeStruct((g, 1), jnp.float32),
        in_specs=[pl.BlockSpec(memory_space=pltpu.SMEM)] +
                 [pl.BlockSpec()] * 6,
    )(ptr, agg2, h2p, degp, b2.reshape(1, h), Wp, bp.reshape(1, 1))
    return out
